# baseline (speedup 1.0000x reference)
import functools

import jax
import jax.numpy as jnp
from jax import lax
from jax.experimental import pallas as pl
from jax.experimental.pallas import tpu as pltpu
from jax.experimental.pallas import tpu_sc as plsc

VEC = 64
CTX = 20
NOISE = 26
BATCH = 16384
LANES = 16
NUM_WORDS = 100000
NUM_DOCS = 100000

_info = plsc.get_sparse_core_info()
NC = _info.num_cores
NS = _info.num_subcores
NW = NC * NS
S = 32
PER_W = BATCH // NW
N_CHUNKS = PER_W // S
TNP = 896
CTXC = S * CTX


@functools.partial(
    pl.kernel,
    mesh=plsc.VectorSubcoreMesh(core_axis_name="c", subcore_axis_name="s"),
    compiler_params=pltpu.CompilerParams(
        needs_layout_passes=False, use_tc_tiling_on_sc=False),
    out_type=jax.ShapeDtypeStruct((BATCH, NOISE), jnp.float32),
    scratch_types=[
        pltpu.VMEM((N_CHUNKS * CTXC,), jnp.int32),
        pltpu.VMEM((PER_W,), jnp.int32),
        pltpu.VMEM((N_CHUNKS * TNP,), jnp.int32),
        pltpu.VMEM((CTXC, VEC), jnp.float32),
        pltpu.VMEM((S, VEC), jnp.float32),
        pltpu.VMEM((TNP, VEC), jnp.float32),
        pltpu.VMEM((S, NOISE), jnp.float32),
        pltpu.SemaphoreType.DMA,
        pltpu.SemaphoreType.DMA,
    ],
)
def _dm_sc(ctx_hbm, doc_hbm, tn_hbm, tab_hbm, out_hbm,
           ctx_idx, doc_idx, tn_idx, wrows, drows, otrows, obuf, sem, osem):
    wid = lax.axis_index("s") * NC + lax.axis_index("c")
    last_lane = lax.iota(jnp.int32, LANES) == (LANES - 1)

    wbase0 = pl.multiple_of(wid * (N_CHUNKS * CTXC), N_CHUNKS * CTXC)
    dbase0 = pl.multiple_of(wid * PER_W, PER_W)
    tbase0 = pl.multiple_of(wid * (N_CHUNKS * TNP), N_CHUNKS * TNP)
    pltpu.sync_copy(ctx_hbm.at[pl.ds(wbase0, N_CHUNKS * CTXC)], ctx_idx)
    pltpu.sync_copy(doc_hbm.at[pl.ds(dbase0, PER_W)], doc_idx)
    pltpu.sync_copy(tn_hbm.at[pl.ds(tbase0, N_CHUNKS * TNP)], tn_idx)

    def chunk_body(c, carry):
        b0 = pl.multiple_of(wid * PER_W + c * S, S)

        cps = []
        for t in range(CTXC // 128):
            cps.append(pltpu.async_copy(
                tab_hbm.at[ctx_idx.at[pl.ds(c * CTXC + t * 128, 128)]],
                wrows.at[pl.ds(t * 128, 128), :], sem))
        cps.append(pltpu.async_copy(
            tab_hbm.at[doc_idx.at[pl.ds(c * S, S)]], drows, sem))
        for t in range(TNP // 128):
            cps.append(pltpu.async_copy(
                tab_hbm.at[tn_idx.at[pl.ds(c * TNP + t * 128, 128)]],
                otrows.at[pl.ds(t * 128, 128), :], sem))
        for cp in cps:
            cp.wait()

        @pl.when(c > 0)
        def _():
            pltpu.make_async_copy(obuf, out_hbm.at[pl.ds(b0 - S, S), :],
                                  osem).wait()

        def row_body(b, carry2):
            x = [drows[b, pl.ds(v * LANES, LANES)] for v in range(VEC // LANES)]
            wb = b * CTX
            for j in range(CTX):
                for v in range(VEC // LANES):
                    x[v] = x[v] + wrows[wb + j, pl.ds(v * LANES, LANES)]
            tb = b * NOISE
            bvec = jnp.full((LANES,), b, jnp.int32)
            for k in range(NOISE):
                p0 = x[0] * otrows[tb + k, pl.ds(0, LANES)]
                p1 = x[1] * otrows[tb + k, pl.ds(LANES, LANES)]
                p2 = x[2] * otrows[tb + k, pl.ds(2 * LANES, LANES)]
                p3 = x[3] * otrows[tb + k, pl.ds(3 * LANES, LANES)]
                cs = plsc.cumsum((p0 + p1) + (p2 + p3))
                plsc.store_scatter(
                    obuf, [bvec, jnp.full((LANES,), k, jnp.int32)], cs,
                    mask=last_lane)
            return carry2

        lax.fori_loop(0, S, row_body, 0)
        pltpu.async_copy(obuf, out_hbm.at[pl.ds(b0, S), :], osem)
        return carry

    lax.fori_loop(0, N_CHUNKS, chunk_body, 0)
    last0 = pl.multiple_of(wid * PER_W + (N_CHUNKS - 1) * S, S)
    pltpu.make_async_copy(obuf, out_hbm.at[pl.ds(last0, S), :], osem).wait()


def kernel(context_ids, doc_ids, target_noise_ids, D, W, O):
    table = jnp.concatenate([W, jnp.transpose(O), D], axis=0)
    ctx_flat = context_ids.reshape(-1).astype(jnp.int32)
    doc_off = doc_ids.astype(jnp.int32) + 2 * NUM_WORDS
    tn2 = target_noise_ids.astype(jnp.int32).reshape(-1, S * NOISE) + NUM_WORDS
    nrow = tn2.shape[0]
    pad_vals = (jnp.arange(nrow, dtype=jnp.int32)[:, None] * (TNP - S * NOISE)
                + jnp.arange(TNP - S * NOISE, dtype=jnp.int32)[None, :]) % NUM_WORDS
    tn_pad = jnp.concatenate([tn2, pad_vals], axis=1).reshape(-1)
    return _dm_sc(ctx_flat, doc_off, tn_pad, table)


# speedup vs baseline: 1.2442x; 1.2442x over previous
import functools

import jax
import jax.numpy as jnp
from jax import lax
from jax.experimental import pallas as pl
from jax.experimental.pallas import tpu as pltpu
from jax.experimental.pallas import tpu_sc as plsc

VEC = 64
CTX = 20
NOISE = 26
BATCH = 16384
LANES = 16

_info = plsc.get_sparse_core_info()
NC = _info.num_cores
NS = _info.num_subcores
NW = NC * NS
S = 32
PER_W = BATCH // NW
N_CHUNKS = PER_W // S
TNP = 896
CTXC = S * CTX


@functools.partial(
    pl.kernel,
    mesh=plsc.VectorSubcoreMesh(core_axis_name="c", subcore_axis_name="s"),
    compiler_params=pltpu.CompilerParams(
        needs_layout_passes=False, use_tc_tiling_on_sc=False),
    out_type=jax.ShapeDtypeStruct((BATCH, NOISE), jnp.float32),
    scratch_types=[
        pltpu.VMEM((N_CHUNKS * CTXC,), jnp.int32),
        pltpu.VMEM((PER_W,), jnp.int32),
        pltpu.VMEM((N_CHUNKS * TNP,), jnp.int32),
        pltpu.VMEM((CTXC, VEC), jnp.float32),
        pltpu.VMEM((S, VEC), jnp.float32),
        pltpu.VMEM((TNP, VEC), jnp.float32),
        pltpu.VMEM((S, NOISE), jnp.float32),
        pltpu.SemaphoreType.DMA,
        pltpu.SemaphoreType.DMA,
    ],
)
def _dm_sc(ctx_hbm, doc_hbm, tn_hbm, d_hbm, w_hbm, ot_hbm, out_hbm,
           ctx_idx, doc_idx, tn_idx, wrows, drows, otrows, obuf, sem, osem):
    wid = lax.axis_index("s") * NC + lax.axis_index("c")
    last_lane = lax.iota(jnp.int32, LANES) == (LANES - 1)

    wbase0 = pl.multiple_of(wid * (N_CHUNKS * CTXC), N_CHUNKS * CTXC)
    dbase0 = pl.multiple_of(wid * PER_W, PER_W)
    tbase0 = pl.multiple_of(wid * (N_CHUNKS * TNP), N_CHUNKS * TNP)
    pltpu.sync_copy(ctx_hbm.at[pl.ds(wbase0, N_CHUNKS * CTXC)], ctx_idx)
    pltpu.sync_copy(doc_hbm.at[pl.ds(dbase0, PER_W)], doc_idx)
    pltpu.sync_copy(tn_hbm.at[pl.ds(tbase0, N_CHUNKS * TNP)], tn_idx)

    def chunk_body(c, carry):
        b0 = pl.multiple_of(wid * PER_W + c * S, S)

        cps = [
            pltpu.async_copy(
                w_hbm.at[ctx_idx.at[pl.ds(c * CTXC, CTXC)]], wrows, sem),
            pltpu.async_copy(
                d_hbm.at[doc_idx.at[pl.ds(c * S, S)]], drows, sem),
            pltpu.async_copy(
                ot_hbm.at[tn_idx.at[pl.ds(c * TNP, TNP)]], otrows, sem),
        ]
        for cp in cps:
            cp.wait()

        @pl.when(c > 0)
        def _():
            pltpu.make_async_copy(obuf, out_hbm.at[pl.ds(b0 - S, S), :],
                                  osem).wait()

        def row_body(b, carry2):
            x = [drows[b, pl.ds(v * LANES, LANES)] for v in range(VEC // LANES)]
            wb = b * CTX
            for j in range(CTX):
                for v in range(VEC // LANES):
                    x[v] = x[v] + wrows[wb + j, pl.ds(v * LANES, LANES)]
            tb = b * NOISE
            bvec = jnp.full((LANES,), b, jnp.int32)
            for k in range(NOISE):
                p0 = x[0] * otrows[tb + k, pl.ds(0, LANES)]
                p1 = x[1] * otrows[tb + k, pl.ds(LANES, LANES)]
                p2 = x[2] * otrows[tb + k, pl.ds(2 * LANES, LANES)]
                p3 = x[3] * otrows[tb + k, pl.ds(3 * LANES, LANES)]
                cs = plsc.cumsum((p0 + p1) + (p2 + p3))
                plsc.store_scatter(
                    obuf, [bvec, jnp.full((LANES,), k, jnp.int32)], cs,
                    mask=last_lane)
            return carry2

        lax.fori_loop(0, S, row_body, 0)
        pltpu.async_copy(obuf, out_hbm.at[pl.ds(b0, S), :], osem)
        return carry

    lax.fori_loop(0, N_CHUNKS, chunk_body, 0)
    last0 = pl.multiple_of(wid * PER_W + (N_CHUNKS - 1) * S, S)
    pltpu.make_async_copy(obuf, out_hbm.at[pl.ds(last0, S), :], osem).wait()


def kernel(context_ids, doc_ids, target_noise_ids, D, W, O):
    ctx_flat = context_ids.reshape(-1).astype(jnp.int32)
    tn2 = target_noise_ids.astype(jnp.int32).reshape(-1, S * NOISE)
    nrow = tn2.shape[0]
    pad_vals = (jnp.arange(nrow, dtype=jnp.int32)[:, None] * (TNP - S * NOISE)
                + jnp.arange(TNP - S * NOISE, dtype=jnp.int32)[None, :]) % 100000
    tn_pad = jnp.concatenate([tn2, pad_vals], axis=1).reshape(-1)
    ot = jnp.transpose(O)
    return _dm_sc(ctx_flat, doc_ids.astype(jnp.int32), tn_pad, D, W, ot)


# speedup vs baseline: 1.4413x; 1.1584x over previous
import functools

import jax
import jax.numpy as jnp
from jax import lax
from jax.experimental import pallas as pl
from jax.experimental.pallas import tpu as pltpu
from jax.experimental.pallas import tpu_sc as plsc

VEC = 64
CTX = 20
NOISE = 26
BATCH = 16384
LANES = 16

_info = plsc.get_sparse_core_info()
NC = _info.num_cores
NS = _info.num_subcores
NW = NC * NS
S = 16
PER_W = BATCH // NW
N_CHUNKS = PER_W // S
CTXP = 384
TNP = 512
DG = 128


@functools.partial(
    pl.kernel,
    mesh=plsc.VectorSubcoreMesh(core_axis_name="c", subcore_axis_name="s"),
    compiler_params=pltpu.CompilerParams(
        needs_layout_passes=False, use_tc_tiling_on_sc=False),
    out_type=jax.ShapeDtypeStruct((BATCH, NOISE), jnp.float32),
    scratch_types=[
        pltpu.VMEM((2 * CTXP,), jnp.int32),
        pltpu.VMEM((PER_W,), jnp.int32),
        pltpu.VMEM((2 * TNP,), jnp.int32),
        pltpu.VMEM((2 * CTXP, VEC), jnp.float32),
        pltpu.VMEM((DG, VEC), jnp.float32),
        pltpu.VMEM((2 * TNP, VEC), jnp.float32),
        pltpu.VMEM((2 * S, NOISE), jnp.float32),
        pltpu.SemaphoreType.DMA,
        pltpu.SemaphoreType.DMA,
        pltpu.SemaphoreType.DMA,
        pltpu.SemaphoreType.DMA,
    ],
)
def _dm_sc(ctx_hbm, doc_hbm, tn_hbm, d_hbm, w_hbm, ot_hbm, out_hbm,
           ctx_idx, doc_idx, tn_idx, wrows, drows, otrows, obuf,
           sem, isem, dsem, osem):
    wid = lax.axis_index("s") * NC + lax.axis_index("c")
    last_lane = lax.iota(jnp.int32, LANES) == (LANES - 1)
    cbase = pl.multiple_of(wid * (N_CHUNKS * CTXP), N_CHUNKS * CTXP)
    tbase = pl.multiple_of(wid * (N_CHUNKS * TNP), N_CHUNKS * TNP)
    obase = pl.multiple_of(wid * PER_W, PER_W)

    def idx_slot(par):
        return (pl.multiple_of(par * CTXP, CTXP),
                pl.multiple_of(par * TNP, TNP))

    def issue_idx_dma(c, par):
        co, to = idx_slot(par)
        pltpu.async_copy(ctx_hbm.at[pl.ds(cbase + c * CTXP, CTXP)],
                         ctx_idx.at[pl.ds(co, CTXP)], isem)
        pltpu.async_copy(tn_hbm.at[pl.ds(tbase + c * TNP, TNP)],
                         tn_idx.at[pl.ds(to, TNP)], isem)

    def wait_idx_dma(c, par):
        co, to = idx_slot(par)
        pltpu.make_async_copy(ctx_hbm.at[pl.ds(cbase + c * CTXP, CTXP)],
                              ctx_idx.at[pl.ds(co, CTXP)], isem).wait()
        pltpu.make_async_copy(tn_hbm.at[pl.ds(tbase + c * TNP, TNP)],
                              tn_idx.at[pl.ds(to, TNP)], isem).wait()

    def gather_ops(par):
        co, to = idx_slot(par)
        gw = pltpu.make_async_copy(
            w_hbm.at[ctx_idx.at[pl.ds(co, CTXP)]],
            wrows.at[pl.ds(co, CTXP), :], sem)
        gt = pltpu.make_async_copy(
            ot_hbm.at[tn_idx.at[pl.ds(to, TNP)]],
            otrows.at[pl.ds(to, TNP), :], sem)
        return gw, gt

    def doc_gather_op(g):
        gb = pl.multiple_of(g * DG, DG)
        return pltpu.make_async_copy(
            d_hbm.at[doc_idx.at[pl.ds(gb, DG)]], drows, dsem)

    def out_copy_op(c, par):
        ob = pl.multiple_of(par * S, S)
        return pltpu.make_async_copy(
            obuf.at[pl.ds(ob, S), :],
            out_hbm.at[pl.ds(obase + c * S, S), :], osem)

    pltpu.sync_copy(doc_hbm.at[pl.ds(obase, PER_W)], doc_idx)
    doc_gather_op(0).start()
    co0, to0 = idx_slot(0)
    pltpu.sync_copy(ctx_hbm.at[pl.ds(cbase, CTXP)], ctx_idx.at[pl.ds(co0, CTXP)])
    pltpu.sync_copy(tn_hbm.at[pl.ds(tbase, TNP)], tn_idx.at[pl.ds(to0, TNP)])
    gw0, gt0 = gather_ops(0)
    gw0.start()
    gt0.start()
    issue_idx_dma(1, 1)

    def chunk_body(c, carry):
        par = lax.rem(c, 2)
        par1 = lax.rem(c + 1, 2)
        rem8 = lax.rem(c, 8)

        @pl.when(rem8 == 0)
        def _():
            doc_gather_op(lax.div(c, 8)).wait()

        gw, gt = gather_ops(par)
        gw.wait()
        gt.wait()

        @pl.when(c >= 2)
        def _():
            out_copy_op(c - 2, par).wait()

        @pl.when(c + 1 < N_CHUNKS)
        def _():
            wait_idx_dma(c + 1, par1)
            ngw, ngt = gather_ops(par1)
            ngw.start()
            ngt.start()

        @pl.when(c + 2 < N_CHUNKS)
        def _():
            issue_idx_dma(c + 2, par)

        wb0 = pl.multiple_of(par * CTXP, CTXP)
        tb0 = pl.multiple_of(par * TNP, TNP)

        def row_body(b, carry2):
            db = rem8 * S + b
            x = [drows[db, pl.ds(v * LANES, LANES)] for v in range(VEC // LANES)]
            wb = wb0 + b * CTX
            for j in range(CTX):
                for v in range(VEC // LANES):
                    x[v] = x[v] + wrows[wb + j, pl.ds(v * LANES, LANES)]
            tb = tb0 + b * NOISE
            bvec = jnp.full((LANES,), par * S + b, jnp.int32)
            for k in range(NOISE):
                p0 = x[0] * otrows[tb + k, pl.ds(0, LANES)]
                p1 = x[1] * otrows[tb + k, pl.ds(LANES, LANES)]
                p2 = x[2] * otrows[tb + k, pl.ds(2 * LANES, LANES)]
                p3 = x[3] * otrows[tb + k, pl.ds(3 * LANES, LANES)]
                cs = plsc.cumsum((p0 + p1) + (p2 + p3))
                plsc.store_scatter(
                    obuf, [bvec, jnp.full((LANES,), k, jnp.int32)], cs,
                    mask=last_lane)
            return carry2

        lax.fori_loop(0, S, row_body, 0)
        out_copy_op(c, par).start()

        @pl.when(jnp.logical_and(rem8 == 7, c + 1 < N_CHUNKS))
        def _():
            doc_gather_op(lax.div(c + 1, 8)).start()

        return carry

    lax.fori_loop(0, N_CHUNKS, chunk_body, 0)
    out_copy_op(N_CHUNKS - 2, 0).wait()
    out_copy_op(N_CHUNKS - 1, 1).wait()


def kernel(context_ids, doc_ids, target_noise_ids, D, W, O):
    nrow = BATCH // S
    ctx2 = context_ids.astype(jnp.int32).reshape(nrow, S * CTX)
    cpad = (jnp.arange(nrow, dtype=jnp.int32)[:, None] * (CTXP - S * CTX)
            + jnp.arange(CTXP - S * CTX, dtype=jnp.int32)[None, :]) % W.shape[0]
    ctx_pad = jnp.concatenate([ctx2, cpad], axis=1).reshape(-1)
    tn2 = target_noise_ids.astype(jnp.int32).reshape(nrow, S * NOISE)
    tpad = (jnp.arange(nrow, dtype=jnp.int32)[:, None] * (TNP - S * NOISE)
            + jnp.arange(TNP - S * NOISE, dtype=jnp.int32)[None, :]) % O.shape[1]
    tn_pad = jnp.concatenate([tn2, tpad], axis=1).reshape(-1)
    ot = jnp.transpose(O)
    return _dm_sc(ctx_pad, doc_ids.astype(jnp.int32), tn_pad, D, W, ot)
